# baseline (speedup 1.0000x reference)
import jax
import jax.numpy as jnp
from jax.experimental import pallas as pl
from jax.experimental.pallas import tpu as pltpu

_N_CHUNKS = 8
_AHEAD = 2
_ROWS = 12288
_CHUNK_ROWS = _ROWS // _N_CHUNKS


def _copy_kernel(in_ref, out_ref, buf, in_sems, out_sems):
    def start_in(i):
        pltpu.make_async_copy(
            in_ref.at[pl.ds(i * _CHUNK_ROWS, _CHUNK_ROWS)], buf.at[i], in_sems.at[i]
        ).start()

    def wait_in(i):
        pltpu.make_async_copy(
            in_ref.at[pl.ds(i * _CHUNK_ROWS, _CHUNK_ROWS)], buf.at[i], in_sems.at[i]
        ).wait()

    def start_out(i):
        pltpu.make_async_copy(
            buf.at[i], out_ref.at[pl.ds(i * _CHUNK_ROWS, _CHUNK_ROWS)], out_sems.at[i]
        ).start()

    def wait_out(i):
        pltpu.make_async_copy(
            buf.at[i], out_ref.at[pl.ds(i * _CHUNK_ROWS, _CHUNK_ROWS)], out_sems.at[i]
        ).wait()

    for i in range(_AHEAD):
        start_in(i)
    for i in range(_N_CHUNKS):
        wait_in(i)
        if i + _AHEAD < _N_CHUNKS:
            start_in(i + _AHEAD)
        start_out(i)
    for i in range(_N_CHUNKS):
        wait_out(i)


def kernel(images):
    flat = images.reshape(_ROWS, 512)
    out = pl.pallas_call(
        _copy_kernel,
        out_shape=jax.ShapeDtypeStruct(flat.shape, flat.dtype),
        in_specs=[pl.BlockSpec(memory_space=pl.ANY)],
        out_specs=pl.BlockSpec(memory_space=pl.ANY),
        scratch_shapes=[
            pltpu.VMEM((_N_CHUNKS, _CHUNK_ROWS, 512), jnp.float32),
            pltpu.SemaphoreType.DMA((_N_CHUNKS,)),
            pltpu.SemaphoreType.DMA((_N_CHUNKS,)),
        ],
    )(flat)
    return out.reshape(images.shape)


# speedup vs baseline: 1.0916x; 1.0916x over previous
import jax
import jax.numpy as jnp
from jax.experimental import pallas as pl
from jax.experimental.pallas import tpu as pltpu

_N_CHUNKS = 4
_AHEAD = 4
_ROWS = 12288
_CHUNK_ROWS = _ROWS // _N_CHUNKS


def _copy_kernel(in_ref, out_ref, buf, in_sems, out_sems):
    def start_in(i):
        pltpu.make_async_copy(
            in_ref.at[pl.ds(i * _CHUNK_ROWS, _CHUNK_ROWS)], buf.at[i], in_sems.at[i]
        ).start()

    def wait_in(i):
        pltpu.make_async_copy(
            in_ref.at[pl.ds(i * _CHUNK_ROWS, _CHUNK_ROWS)], buf.at[i], in_sems.at[i]
        ).wait()

    def start_out(i):
        pltpu.make_async_copy(
            buf.at[i], out_ref.at[pl.ds(i * _CHUNK_ROWS, _CHUNK_ROWS)], out_sems.at[i]
        ).start()

    def wait_out(i):
        pltpu.make_async_copy(
            buf.at[i], out_ref.at[pl.ds(i * _CHUNK_ROWS, _CHUNK_ROWS)], out_sems.at[i]
        ).wait()

    for i in range(_AHEAD):
        start_in(i)
    for i in range(_N_CHUNKS):
        wait_in(i)
        if i + _AHEAD < _N_CHUNKS:
            start_in(i + _AHEAD)
        start_out(i)
    for i in range(_N_CHUNKS):
        wait_out(i)


def kernel(images):
    flat = images.reshape(_ROWS, 512)
    out = pl.pallas_call(
        _copy_kernel,
        out_shape=jax.ShapeDtypeStruct(flat.shape, flat.dtype),
        in_specs=[pl.BlockSpec(memory_space=pl.ANY)],
        out_specs=pl.BlockSpec(memory_space=pl.ANY),
        scratch_shapes=[
            pltpu.VMEM((_N_CHUNKS, _CHUNK_ROWS, 512), jnp.float32),
            pltpu.SemaphoreType.DMA((_N_CHUNKS,)),
            pltpu.SemaphoreType.DMA((_N_CHUNKS,)),
        ],
    )(flat)
    return out.reshape(images.shape)


# speedup vs baseline: 1.1060x; 1.0131x over previous
import jax
import jax.numpy as jnp
from jax.experimental import pallas as pl
from jax.experimental.pallas import tpu as pltpu

_N_CHUNKS = 2
_AHEAD = 2
_ROWS = 12288
_CHUNK_ROWS = _ROWS // _N_CHUNKS


def _copy_kernel(in_ref, out_ref, buf, in_sems, out_sems):
    def start_in(i):
        pltpu.make_async_copy(
            in_ref.at[pl.ds(i * _CHUNK_ROWS, _CHUNK_ROWS)], buf.at[i], in_sems.at[i]
        ).start()

    def wait_in(i):
        pltpu.make_async_copy(
            in_ref.at[pl.ds(i * _CHUNK_ROWS, _CHUNK_ROWS)], buf.at[i], in_sems.at[i]
        ).wait()

    def start_out(i):
        pltpu.make_async_copy(
            buf.at[i], out_ref.at[pl.ds(i * _CHUNK_ROWS, _CHUNK_ROWS)], out_sems.at[i]
        ).start()

    def wait_out(i):
        pltpu.make_async_copy(
            buf.at[i], out_ref.at[pl.ds(i * _CHUNK_ROWS, _CHUNK_ROWS)], out_sems.at[i]
        ).wait()

    for i in range(_AHEAD):
        start_in(i)
    for i in range(_N_CHUNKS):
        wait_in(i)
        if i + _AHEAD < _N_CHUNKS:
            start_in(i + _AHEAD)
        start_out(i)
    for i in range(_N_CHUNKS):
        wait_out(i)


def kernel(images):
    flat = images.reshape(_ROWS, 512)
    out = pl.pallas_call(
        _copy_kernel,
        out_shape=jax.ShapeDtypeStruct(flat.shape, flat.dtype),
        in_specs=[pl.BlockSpec(memory_space=pl.ANY)],
        out_specs=pl.BlockSpec(memory_space=pl.ANY),
        scratch_shapes=[
            pltpu.VMEM((_N_CHUNKS, _CHUNK_ROWS, 512), jnp.float32),
            pltpu.SemaphoreType.DMA((_N_CHUNKS,)),
            pltpu.SemaphoreType.DMA((_N_CHUNKS,)),
        ],
    )(flat)
    return out.reshape(images.shape)


# speedup vs baseline: 1.1149x; 1.0081x over previous
import jax
import jax.numpy as jnp
from jax.experimental import pallas as pl
from jax.experimental.pallas import tpu as pltpu

_ROWS_PER_BLOCK = 6144


def _copy_kernel(in_ref, out_ref):
    out_ref[...] = in_ref[...]


def kernel(images):
    flat = images.reshape(-1, 512)
    rows = flat.shape[0]
    grid = rows // _ROWS_PER_BLOCK
    out = pl.pallas_call(
        _copy_kernel,
        grid=(grid,),
        in_specs=[pl.BlockSpec((_ROWS_PER_BLOCK, 512), lambda i: (i, 0))],
        out_specs=pl.BlockSpec((_ROWS_PER_BLOCK, 512), lambda i: (i, 0)),
        out_shape=jax.ShapeDtypeStruct(flat.shape, flat.dtype),
        compiler_params=pltpu.CompilerParams(
            dimension_semantics=("parallel",),
        ),
    )(flat)
    return out.reshape(images.shape)
